# baseline (speedup 1.0000x reference)
import functools

import jax
import jax.numpy as jnp
from jax import lax
from jax.experimental import pallas as pl
from jax.experimental.pallas import tpu as pltpu
from jax.experimental.pallas import tpu_sc as plsc

N_NODES = 10000
N_EDGES = 320000
D_IN = 128
D_HID = 128
D_EDGE = 16
NUM_GRAPHS = 8

NC = 2
NS = 16
NT = NC * NS
EPT = N_EDGES // NT
K = 128
CH = 80
SLAB = 64
EPAD = CH * K - EPT
NROWS = N_NODES + 8
TRASH = N_NODES
RPT = 624
TAIL = N_NODES - NS * RPT
DEA = 128

_f32 = jnp.float32


@functools.cache
def _get_mesh():
    return plsc.VectorSubcoreMesh(core_axis_name="c", subcore_axis_name="s",
                                  num_cores=NC, num_subcores=NS)


def _zero_acc(acc, z_hbm, s):
    pltpu.sync_copy(z_hbm.at[pl.ds(0, RPT)], acc.at[pl.ds(s * RPT, RPT)])

    @pl.when(s == NS - 1)
    def _():
        pltpu.sync_copy(z_hbm.at[pl.ds(0, TAIL + 8)],
                        acc.at[pl.ds(NS * RPT, TAIL + 8)])


def _writeback(acc, out_hbm, c, s):
    pltpu.sync_copy(acc.at[pl.ds(s * RPT, RPT)],
                    out_hbm.at[c, pl.ds(s * RPT, RPT)])

    @pl.when(s == NS - 1)
    def _():
        pltpu.sync_copy(acc.at[pl.ds(NS * RPT, TAIL)],
                        out_hbm.at[c, pl.ds(NS * RPT, TAIL)])


def _dbuf_phase(n, load, scat, wait_g, wait_s, r0, r1):
    load(0, r0)
    wait_g(r0)
    load(1, r1)
    scat(0, r0)

    def pair(i, carry):
        j0 = 2 * i + 1
        wait_g(r1)
        wait_s(r0)
        load(j0 + 1, r0)
        scat(j0, r1)
        j1 = j0 + 1
        wait_g(r0)
        wait_s(r1)

        @pl.when(j1 + 1 < n)
        def _():
            load(j1 + 1, r1)

        scat(j1, r0)
        return carry

    lax.fori_loop(0, (n - 2) // 2, pair, 0)
    wait_g(r1)
    wait_s(r0)
    scat(n - 1, r1)


def _dbuf_loop(load_g, stage_idx, dst_v, acc, z_hbm, r0, r1, semg, sems):
    def wait_g(buf):
        pltpu.make_async_copy(z_hbm.at[pl.ds(0, K)], buf, semg).wait()

    def wait_s(buf):
        pltpu.make_async_copy(z_hbm.at[pl.ds(0, K)], buf, sems).wait()

    first = True
    for base in range(0, CH, SLAB):
        n = min(SLAB, CH - base)
        if not first:
            wait_s(r1)
        stage_idx(base, n)

        def load(j, buf, base=base):
            load_g(base + j, buf)

        def scat(j, buf):
            pltpu.async_copy(buf, acc.at[dst_v.at[j]], sems, add=True)

        _dbuf_phase(n, load, scat, wait_g, wait_s, r0, r1)
        first = False
    wait_s(r1)


def _sc_gather_sum_body(h_hbm, srcr_hbm, dstr_hbm, zg_hbm, outg_hbm,
                        src_v, dst_v, r0, r1, accg, semg, sems):
    c = lax.axis_index("c")
    s = lax.axis_index("s")
    wid = c * NS + s
    _zero_acc(accg, zg_hbm, s)
    plsc.subcore_barrier()

    def stage_idx(base, n):
        pltpu.sync_copy(srcr_hbm.at[wid, pl.ds(base, n)], src_v.at[pl.ds(0, n)])
        pltpu.sync_copy(dstr_hbm.at[wid, pl.ds(base, n)], dst_v.at[pl.ds(0, n)])

    def load(j, buf):
        pltpu.async_copy(h_hbm.at[src_v.at[j % SLAB]], buf, semg)

    def serial(base, n):
        def body(j, carry):
            pltpu.async_copy(h_hbm.at[src_v.at[j]], r0, semg).wait()
            pltpu.sync_copy(r0, accg.at[dst_v.at[j]], add=True)
            return carry
        lax.fori_loop(0, n, body, 0)

    for base in range(0, CH, SLAB):
        n = min(SLAB, CH - base)
        stage_idx(base, n)
        serial(base, n)

    plsc.subcore_barrier()
    _writeback(accg, outg_hbm, c, s)


@functools.cache
def _build_sc_gather_sum():
    return pl.kernel(
        _sc_gather_sum_body,
        out_type=jax.ShapeDtypeStruct((NC, N_NODES, D_HID), _f32),
        mesh=_get_mesh(),
        scratch_types=[
            pltpu.VMEM((SLAB, K), jnp.int32),
            pltpu.VMEM((SLAB, K), jnp.int32),
            pltpu.VMEM((K, D_HID), _f32),
            pltpu.VMEM((K, D_HID), _f32),
            pltpu.VMEM_SHARED((NROWS, D_HID), _f32),
            pltpu.SemaphoreType.DMA,
            pltpu.SemaphoreType.DMA,
        ],
    )


def _sc_edge_sum_body(ea_hbm, dstr_hbm, ze_hbm, oute_hbm,
                      dst_v, r0, r1, acce, semg, sems):
    c = lax.axis_index("c")
    s = lax.axis_index("s")
    wid = c * NS + s
    _zero_acc(acce, ze_hbm, s)
    plsc.subcore_barrier()

    def stage_idx(base, n):
        pltpu.sync_copy(dstr_hbm.at[wid, pl.ds(base, n)], dst_v.at[pl.ds(0, n)])

    def load(j, buf):
        pltpu.async_copy(ea_hbm.at[wid, pl.ds(j * K, K)], buf, semg)

    _dbuf_loop(load, stage_idx, dst_v, acce, ze_hbm, r0, r1, semg, sems)
    plsc.subcore_barrier()
    _writeback(acce, oute_hbm, c, s)


@functools.cache
def _build_sc_edge_sum():
    return pl.kernel(
        _sc_edge_sum_body,
        out_type=jax.ShapeDtypeStruct((NC, N_NODES, DEA), _f32),
        mesh=_get_mesh(),
        scratch_types=[
            pltpu.VMEM((SLAB, K), jnp.int32),
            pltpu.VMEM((K, DEA), _f32),
            pltpu.VMEM((K, DEA), _f32),
            pltpu.VMEM_SHARED((NROWS, DEA), _f32),
            pltpu.SemaphoreType.DMA,
            pltpu.SemaphoreType.DMA,
        ],
    )


def _dot(a, b):
    return lax.dot_general(a, b, (((1,), (0,)), ((), ())),
                           precision=lax.Precision.HIGHEST,
                           preferred_element_type=_f32)


def _tc_layer_kernel(gp_ref, ep_ref, hin_ref, wx_ref, we_ref, bn_ref,
                     wr_ref, br_ref, g_ref, b_ref, out_ref):
    gsum = gp_ref[0] + gp_ref[1]
    esum = ep_ref[0] + ep_ref[1]
    cnt = esum[:, D_EDGE:D_EDGE + 1]
    msum = (_dot(gsum, wx_ref[0]) + _dot(esum[:, :D_EDGE], we_ref[0])
            + cnt * bn_ref[...])
    agg = msum / jnp.maximum(cnt, 1.0)
    h = agg + _dot(hin_ref[...], wr_ref[0]) + br_ref[...]
    h = jnp.maximum(h, 0.0)
    mu = jnp.mean(h, axis=-1, keepdims=True)
    var = jnp.mean((h - mu) ** 2, axis=-1, keepdims=True)
    out_ref[...] = (h - mu) * lax.rsqrt(var + 1e-5) * g_ref[...] + b_ref[...]


def _tc_pool_kernel(h_ref, wp_ref, bp_ref, ws_ref, bs_ref, batch_ref,
                    out_ref):
    h = h_ref[...]
    hp = jnp.tanh(_dot(h, wp_ref[0]) + bp_ref[...])
    s = _dot(hp, ws_ref[...]) + bs_ref[...]
    gid = lax.broadcasted_iota(jnp.int32, (N_NODES, NUM_GRAPHS), 1)
    oh = (batch_ref[...] == gid)
    ohf = oh.astype(_f32)
    smax = jnp.max(jnp.where(oh, s, -1e30), axis=0, keepdims=True)
    smax_b = jnp.sum(jnp.where(oh, smax, 0.0), axis=1, keepdims=True)
    e = jnp.exp(s - smax_b)
    denom = jnp.sum(ohf * e, axis=0, keepdims=True)
    denom_b = jnp.sum(ohf * denom, axis=1, keepdims=True)
    w = e / denom_b
    out_ref[...] = lax.dot_general(
        ohf * w, h, (((0,), (0,)), ((), ())),
        precision=lax.Precision.HIGHEST, preferred_element_type=_f32)


_RB = 1000
_NRB = N_NODES // _RB


def _tc_layer(gp, ep, hin, wn, bn, wr, br, g, b):
    return pl.pallas_call(
        _tc_layer_kernel,
        grid=(_NRB,),
        in_specs=[
            pl.BlockSpec((NC, _RB, D_HID), lambda i: (0, i, 0)),
            pl.BlockSpec((NC, _RB, DEA), lambda i: (0, i, 0)),
            pl.BlockSpec((_RB, D_HID), lambda i: (i, 0)),
            pl.BlockSpec((1, D_HID, D_HID), lambda i: (0, 0, 0)),
            pl.BlockSpec((1, D_EDGE, D_HID), lambda i: (0, 0, 0)),
            pl.BlockSpec((1, D_HID), lambda i: (0, 0)),
            pl.BlockSpec((1, D_HID, D_HID), lambda i: (0, 0, 0)),
            pl.BlockSpec((1, D_HID), lambda i: (0, 0)),
            pl.BlockSpec((1, D_HID), lambda i: (0, 0)),
            pl.BlockSpec((1, D_HID), lambda i: (0, 0)),
        ],
        out_specs=pl.BlockSpec((_RB, D_HID), lambda i: (i, 0)),
        out_shape=jax.ShapeDtypeStruct((N_NODES, D_HID), _f32),
    )(gp, ep, hin,
      wn[:D_HID].reshape(1, D_HID, D_HID),
      wn[D_HID:].reshape(1, D_EDGE, D_HID),
      bn.reshape(1, D_HID),
      wr.reshape(1, D_HID, D_HID),
      br.reshape(1, D_HID),
      g.reshape(1, D_HID), b.reshape(1, D_HID))


def kernel(x, edge_index, edge_attr, batch,
           W_neigh0, b_neigh0, W_root0, b_root0, ln_g0, ln_b0,
           W_neigh1, b_neigh1, W_root1, b_root1, ln_g1, ln_b1,
           W_proj, b_proj, w_score, b_score):
    i32 = jnp.int32
    src_r = jnp.concatenate(
        [edge_index[0].reshape(NT, EPT),
         jnp.zeros((NT, EPAD), i32)], axis=1).reshape(NT, CH, K)
    dst_r = jnp.concatenate(
        [edge_index[1].reshape(NT, EPT),
         jnp.full((NT, EPAD), TRASH, i32)], axis=1).reshape(NT, CH, K)
    ea_pad = jnp.concatenate(
        [edge_attr.reshape(NT, EPT, D_EDGE),
         jnp.ones((NT, EPT, 1), _f32),
         jnp.zeros((NT, EPT, DEA - D_EDGE - 1), _f32)], axis=2)
    ea_pad = jnp.concatenate(
        [ea_pad, jnp.zeros((NT, EPAD, DEA), _f32)], axis=1)
    zg = jnp.zeros((RPT, D_HID), _f32)

    gather_sum = _build_sc_gather_sum()
    ep = _build_sc_edge_sum()(ea_pad, dst_r, zg)
    gp0 = gather_sum(x, src_r, dst_r, zg)
    h0 = _tc_layer(gp0, ep, x, W_neigh0, b_neigh0, W_root0, b_root0,
                   ln_g0, ln_b0)
    gp1 = gather_sum(h0, src_r, dst_r, zg)
    h1 = _tc_layer(gp1, ep, h0, W_neigh1, b_neigh1, W_root1, b_root1,
                   ln_g1, ln_b1)

    pooled = pl.pallas_call(
        _tc_pool_kernel,
        out_shape=jax.ShapeDtypeStruct((NUM_GRAPHS, D_HID), _f32),
    )(h1, W_proj.reshape(1, D_HID, D_HID), b_proj.reshape(1, D_HID),
      w_score.reshape(D_HID, 1), b_score.reshape(1, 1),
      batch.reshape(N_NODES, 1))
    return pooled


# speedup vs baseline: 1.4432x; 1.4432x over previous
import functools

import jax
import jax.numpy as jnp
from jax import lax
from jax.experimental import pallas as pl
from jax.experimental.pallas import tpu as pltpu
from jax.experimental.pallas import tpu_sc as plsc

N_NODES = 10000
N_EDGES = 320000
D_IN = 128
D_HID = 128
D_EDGE = 16
NUM_GRAPHS = 8

NC = 2
NS = 16
NT = NC * NS
EPT = N_EDGES // NT
KG = 80
CHG = 126
KE = 128
CHE = 80
SLAB = 64
NROWS = N_NODES + NT
RPT = 624
TAIL = N_NODES - NS * RPT
DEA = 128

_f32 = jnp.float32


@functools.cache
def _get_mesh():
    return plsc.VectorSubcoreMesh(core_axis_name="c", subcore_axis_name="s",
                                  num_cores=NC, num_subcores=NS)


def _zero_acc(acc, z_hbm, s):
    pltpu.sync_copy(z_hbm.at[pl.ds(0, RPT)], acc.at[pl.ds(s * RPT, RPT)])

    @pl.when(s == NS - 1)
    def _():
        pltpu.sync_copy(z_hbm.at[pl.ds(0, TAIL + NT)],
                        acc.at[pl.ds(NS * RPT, TAIL + NT)])


def _writeback(acc, out_hbm, c, s):
    pltpu.sync_copy(acc.at[pl.ds(s * RPT, RPT)],
                    out_hbm.at[c, pl.ds(s * RPT, RPT)])

    @pl.when(s == NS - 1)
    def _():
        pltpu.sync_copy(acc.at[pl.ds(NS * RPT, TAIL)],
                        out_hbm.at[c, pl.ds(NS * RPT, TAIL)])


def _dbuf_phase(n, load, scat, wait_g, wait_s, r0, r1):
    load(0, r0)
    wait_g(r0)
    load(1, r1)
    scat(0, r0)

    def pair(i, carry):
        j0 = 2 * i + 1
        wait_g(r1)
        wait_s(r0)
        load(j0 + 1, r0)
        scat(j0, r1)
        j1 = j0 + 1
        wait_g(r0)
        wait_s(r1)

        @pl.when(j1 + 1 < n)
        def _():
            load(j1 + 1, r1)

        scat(j1, r0)
        return carry

    lax.fori_loop(0, (n - 2) // 2, pair, 0)
    wait_g(r1)
    wait_s(r0)
    scat(n - 1, r1)


def _dbuf_loop(kk, ch, load_g, stage_idx, dst_v, acc, z_hbm,
               r0, r1, semg, sems):
    def wait_g(buf):
        pltpu.make_async_copy(z_hbm.at[pl.ds(0, kk)], buf, semg).wait()

    def wait_s(buf):
        pltpu.make_async_copy(z_hbm.at[pl.ds(0, kk)], buf, sems).wait()

    first = True
    for base in range(0, ch, SLAB):
        n = min(SLAB, ch - base)
        if not first:
            wait_s(r1)
        stage_idx(base, n)

        def load(j, buf, base=base):
            load_g(base + j, buf)

        def scat(j, buf):
            pltpu.async_copy(buf, acc.at[dst_v.at[j]], sems, add=True)

        _dbuf_phase(n, load, scat, wait_g, wait_s, r0, r1)
        first = False
    wait_s(r1)


def _sc_gather_sum_body(h_hbm, srcr_hbm, dstr_hbm, zg_hbm, outg_hbm,
                        src_v, dst_v, r0, r1, accg, semg, sems):
    c = lax.axis_index("c")
    s = lax.axis_index("s")
    wid = c * NS + s
    _zero_acc(accg, zg_hbm, s)
    plsc.subcore_barrier()

    def stage_idx(base, n):
        pltpu.sync_copy(srcr_hbm.at[wid, pl.ds(base, n)],
                        src_v.at[pl.ds(0, n)])
        pltpu.sync_copy(dstr_hbm.at[wid, pl.ds(base, n)],
                        dst_v.at[pl.ds(0, n)])

    def load(j, buf):
        pltpu.async_copy(h_hbm.at[src_v.at[j % SLAB]], buf, semg)

    _dbuf_loop(KG, CHG, load, stage_idx, dst_v, accg, zg_hbm,
               r0, r1, semg, sems)
    plsc.subcore_barrier()
    _writeback(accg, outg_hbm, c, s)


@functools.cache
def _build_sc_gather_sum():
    return pl.kernel(
        _sc_gather_sum_body,
        out_type=jax.ShapeDtypeStruct((NC, N_NODES, D_HID), _f32),
        mesh=_get_mesh(),
        scratch_types=[
            pltpu.VMEM((SLAB, KG), jnp.int32),
            pltpu.VMEM((SLAB, KG), jnp.int32),
            pltpu.VMEM((KG, D_HID), _f32),
            pltpu.VMEM((KG, D_HID), _f32),
            pltpu.VMEM_SHARED((NROWS, D_HID), _f32),
            pltpu.SemaphoreType.DMA,
            pltpu.SemaphoreType.DMA,
        ],
    )


def _sc_edge_sum_body(ea_hbm, dstr_hbm, ze_hbm, oute_hbm,
                      dst_v, r0, r1, acce, semg, sems):
    c = lax.axis_index("c")
    s = lax.axis_index("s")
    wid = c * NS + s
    _zero_acc(acce, ze_hbm, s)
    plsc.subcore_barrier()

    def stage_idx(base, n):
        pltpu.sync_copy(dstr_hbm.at[wid, pl.ds(base, n)],
                        dst_v.at[pl.ds(0, n)])

    def load(j, buf):
        pltpu.async_copy(ea_hbm.at[wid, pl.ds(j * KE, KE)], buf, semg)

    _dbuf_loop(KE, CHE, load, stage_idx, dst_v, acce, ze_hbm,
               r0, r1, semg, sems)
    plsc.subcore_barrier()
    _writeback(acce, oute_hbm, c, s)


@functools.cache
def _build_sc_edge_sum():
    return pl.kernel(
        _sc_edge_sum_body,
        out_type=jax.ShapeDtypeStruct((NC, N_NODES, DEA), _f32),
        mesh=_get_mesh(),
        scratch_types=[
            pltpu.VMEM((SLAB, KE), jnp.int32),
            pltpu.VMEM((KE, DEA), _f32),
            pltpu.VMEM((KE, DEA), _f32),
            pltpu.VMEM_SHARED((NROWS, DEA), _f32),
            pltpu.SemaphoreType.DMA,
            pltpu.SemaphoreType.DMA,
        ],
    )


def _dot(a, b):
    return lax.dot_general(a, b, (((1,), (0,)), ((), ())),
                           precision=lax.Precision.HIGHEST,
                           preferred_element_type=_f32)


def _tc_layer_kernel(gp_ref, ep_ref, hin_ref, wx_ref, we_ref, bn_ref,
                     wr_ref, br_ref, g_ref, b_ref, out_ref):
    gsum = gp_ref[0] + gp_ref[1]
    esum = ep_ref[0] + ep_ref[1]
    cnt = esum[:, D_EDGE:D_EDGE + 1]
    msum = (_dot(gsum, wx_ref[0]) + _dot(esum[:, :D_EDGE], we_ref[0])
            + cnt * bn_ref[...])
    agg = msum / jnp.maximum(cnt, 1.0)
    h = agg + _dot(hin_ref[...], wr_ref[0]) + br_ref[...]
    h = jnp.maximum(h, 0.0)
    mu = jnp.mean(h, axis=-1, keepdims=True)
    var = jnp.mean((h - mu) ** 2, axis=-1, keepdims=True)
    out_ref[...] = (h - mu) * lax.rsqrt(var + 1e-5) * g_ref[...] + b_ref[...]


def _tc_pool_kernel(h_ref, wp_ref, bp_ref, ws_ref, bs_ref, batch_ref,
                    out_ref):
    h = h_ref[...]
    hp = jnp.tanh(_dot(h, wp_ref[0]) + bp_ref[...])
    s = _dot(hp, ws_ref[...]) + bs_ref[...]
    gid = lax.broadcasted_iota(jnp.int32, (N_NODES, NUM_GRAPHS), 1)
    oh = (batch_ref[...] == gid)
    ohf = oh.astype(_f32)
    smax = jnp.max(jnp.where(oh, s, -1e30), axis=0, keepdims=True)
    smax_b = jnp.sum(jnp.where(oh, smax, 0.0), axis=1, keepdims=True)
    e = jnp.exp(s - smax_b)
    denom = jnp.sum(ohf * e, axis=0, keepdims=True)
    denom_b = jnp.sum(ohf * denom, axis=1, keepdims=True)
    w = e / denom_b
    out_ref[...] = lax.dot_general(
        ohf * w, h, (((0,), (0,)), ((), ())),
        precision=lax.Precision.HIGHEST, preferred_element_type=_f32)


_RB = 1000
_NRB = N_NODES // _RB


def _tc_layer(gp, ep, hin, wn, bn, wr, br, g, b):
    return pl.pallas_call(
        _tc_layer_kernel,
        grid=(_NRB,),
        in_specs=[
            pl.BlockSpec((NC, _RB, D_HID), lambda i: (0, i, 0)),
            pl.BlockSpec((NC, _RB, DEA), lambda i: (0, i, 0)),
            pl.BlockSpec((_RB, D_HID), lambda i: (i, 0)),
            pl.BlockSpec((1, D_HID, D_HID), lambda i: (0, 0, 0)),
            pl.BlockSpec((1, D_EDGE, D_HID), lambda i: (0, 0, 0)),
            pl.BlockSpec((1, D_HID), lambda i: (0, 0)),
            pl.BlockSpec((1, D_HID, D_HID), lambda i: (0, 0, 0)),
            pl.BlockSpec((1, D_HID), lambda i: (0, 0)),
            pl.BlockSpec((1, D_HID), lambda i: (0, 0)),
            pl.BlockSpec((1, D_HID), lambda i: (0, 0)),
        ],
        out_specs=pl.BlockSpec((_RB, D_HID), lambda i: (i, 0)),
        out_shape=jax.ShapeDtypeStruct((N_NODES, D_HID), _f32),
    )(gp, ep, hin,
      wn[:D_HID].reshape(1, D_HID, D_HID),
      wn[D_HID:].reshape(1, D_EDGE, D_HID),
      bn.reshape(1, D_HID),
      wr.reshape(1, D_HID, D_HID),
      br.reshape(1, D_HID),
      g.reshape(1, D_HID), b.reshape(1, D_HID))


def _pad_idx(idx, ch, kk, trash):
    n_pad = ch * kk - EPT
    body = idx.reshape(NT, EPT)
    if trash:
        fill = (N_NODES + jnp.arange(NT, dtype=jnp.int32))[:, None]
        fill = jnp.broadcast_to(fill, (NT, n_pad))
    else:
        fill = jnp.zeros((NT, n_pad), jnp.int32)
    return jnp.concatenate([body, fill], axis=1).reshape(NT, ch, kk)


def kernel(x, edge_index, edge_attr, batch,
           W_neigh0, b_neigh0, W_root0, b_root0, ln_g0, ln_b0,
           W_neigh1, b_neigh1, W_root1, b_root1, ln_g1, ln_b1,
           W_proj, b_proj, w_score, b_score):
    src_g = _pad_idx(edge_index[0], CHG, KG, trash=False)
    dst_g = _pad_idx(edge_index[1], CHG, KG, trash=True)
    dst_e = _pad_idx(edge_index[1], CHE, KE, trash=True)
    ea_pad = jnp.concatenate(
        [edge_attr.reshape(NT, EPT, D_EDGE),
         jnp.ones((NT, EPT, 1), _f32),
         jnp.zeros((NT, EPT, DEA - D_EDGE - 1), _f32)], axis=2)
    ea_pad = jnp.concatenate(
        [ea_pad, jnp.zeros((NT, CHE * KE - EPT, DEA), _f32)], axis=1)
    zg = jnp.zeros((RPT, D_HID), _f32)

    gather_sum = _build_sc_gather_sum()
    ep = _build_sc_edge_sum()(ea_pad, dst_e, zg)
    gp0 = gather_sum(x, src_g, dst_g, zg)
    h0 = _tc_layer(gp0, ep, x, W_neigh0, b_neigh0, W_root0, b_root0,
                   ln_g0, ln_b0)
    gp1 = gather_sum(h0, src_g, dst_g, zg)
    h1 = _tc_layer(gp1, ep, h0, W_neigh1, b_neigh1, W_root1, b_root1,
                   ln_g1, ln_b1)

    pooled = pl.pallas_call(
        _tc_pool_kernel,
        out_shape=jax.ShapeDtypeStruct((NUM_GRAPHS, D_HID), _f32),
    )(h1, W_proj.reshape(1, D_HID, D_HID), b_proj.reshape(1, D_HID),
      w_score.reshape(D_HID, 1), b_score.reshape(1, 1),
      batch.reshape(N_NODES, 1))
    return pooled


# speedup vs baseline: 1.6289x; 1.1287x over previous
import functools

import jax
import jax.numpy as jnp
from jax import lax
from jax.experimental import pallas as pl
from jax.experimental.pallas import tpu as pltpu
from jax.experimental.pallas import tpu_sc as plsc

N_NODES = 10000
N_EDGES = 320000
D_IN = 128
D_HID = 128
D_EDGE = 16
NUM_GRAPHS = 8

NC = 2
NS = 16
NT = NC * NS
EPT = N_EDGES // NT
KG = 80
CHG = 125
KE = 128
CHE = 80
SLAB = 64
NROWS = N_NODES + NT
RPT = 624
TAIL = N_NODES - NS * RPT
DEA = 128

_f32 = jnp.float32


@functools.cache
def _get_mesh():
    return plsc.VectorSubcoreMesh(core_axis_name="c", subcore_axis_name="s",
                                  num_cores=NC, num_subcores=NS)


def _zero_acc(acc, z_hbm, s):
    pltpu.sync_copy(z_hbm.at[pl.ds(0, RPT)], acc.at[pl.ds(s * RPT, RPT)])

    @pl.when(s == NS - 1)
    def _():
        pltpu.sync_copy(z_hbm.at[pl.ds(0, TAIL + NT)],
                        acc.at[pl.ds(NS * RPT, TAIL + NT)])


def _writeback(acc, out_hbm, c, s):
    pltpu.sync_copy(acc.at[pl.ds(s * RPT, RPT)],
                    out_hbm.at[c, pl.ds(s * RPT, RPT)])

    @pl.when(s == NS - 1)
    def _():
        pltpu.sync_copy(acc.at[pl.ds(NS * RPT, TAIL)],
                        out_hbm.at[c, pl.ds(NS * RPT, TAIL)])


def _dbuf_phase(n, load, scat, wait_g, wait_s, r0, r1):
    load(0, r0)
    wait_g(r0)
    load(1, r1)
    scat(0, r0)

    def pair(i, carry):
        j0 = 2 * i + 1
        wait_g(r1)
        wait_s(r0)
        load(j0 + 1, r0)
        scat(j0, r1)
        j1 = j0 + 1
        wait_g(r0)
        wait_s(r1)

        @pl.when(j1 + 1 < n)
        def _():
            load(j1 + 1, r1)

        scat(j1, r0)
        return carry

    lax.fori_loop(0, (n - 2) // 2, pair, 0)
    wait_g(r1)
    wait_s(r0)
    scat(n - 1, r1)


def _dbuf_loop(kk, ch, load_g, stage_idx, dst_v, acc, z_hbm,
               r0, r1, semg, sems):
    def wait_g(buf):
        pltpu.make_async_copy(z_hbm.at[pl.ds(0, kk)], buf, semg).wait()

    def wait_s(buf):
        pltpu.make_async_copy(z_hbm.at[pl.ds(0, kk)], buf, sems).wait()

    first = True
    for base in range(0, ch, SLAB):
        n = min(SLAB, ch - base)
        if not first:
            wait_s(r1)
        stage_idx(base, n)

        def load(j, buf, base=base):
            load_g(base + j, buf)

        def scat(j, buf):
            pltpu.async_copy(buf, acc.at[dst_v.at[j]], sems, add=True)

        _dbuf_phase(n, load, scat, wait_g, wait_s, r0, r1)
        first = False
    wait_s(r1)


def _sc_gather_sum_body(h_hbm, srcr_hbm, dstr_hbm, zg_hbm, outg_hbm,
                        src_v, dst_v, rows_v, accg, sem):
    c = lax.axis_index("c")
    s = lax.axis_index("s")
    wid = c * NS + s
    pltpu.sync_copy(zg_hbm.at[pl.ds(0, RPT)], accg.at[pl.ds(s * RPT, RPT)])

    @pl.when(s == NS - 1)
    def _():
        pltpu.sync_copy(zg_hbm.at[pl.ds(0, TAIL)],
                        accg.at[pl.ds(NS * RPT, TAIL)])

    pltpu.sync_copy(srcr_hbm.at[wid], src_v)
    pltpu.sync_copy(dstr_hbm.at[wid], dst_v)
    plsc.subcore_barrier()

    def body(j, carry):
        pltpu.async_copy(h_hbm.at[src_v.at[j]], rows_v, sem).wait()
        pltpu.sync_copy(rows_v, accg.at[dst_v.at[j]], add=True)
        return carry

    lax.fori_loop(0, CHG, body, 0)
    plsc.subcore_barrier()
    _writeback(accg, outg_hbm, c, s)


@functools.cache
def _build_sc_gather_sum():
    return pl.kernel(
        _sc_gather_sum_body,
        out_type=jax.ShapeDtypeStruct((NC, N_NODES, D_HID), _f32),
        mesh=_get_mesh(),
        scratch_types=[
            pltpu.VMEM((CHG, KG), jnp.int32),
            pltpu.VMEM((CHG, KG), jnp.int32),
            pltpu.VMEM((KG, D_HID), _f32),
            pltpu.VMEM_SHARED((N_NODES, D_HID), _f32),
            pltpu.SemaphoreType.DMA,
        ],
    )


def _sc_edge_sum_body(ea_hbm, dstr_hbm, ze_hbm, oute_hbm,
                      dst_v, r0, r1, acce, semg, sems):
    c = lax.axis_index("c")
    s = lax.axis_index("s")
    wid = c * NS + s
    _zero_acc(acce, ze_hbm, s)
    plsc.subcore_barrier()

    def stage_idx(base, n):
        pltpu.sync_copy(dstr_hbm.at[wid, pl.ds(base, n)],
                        dst_v.at[pl.ds(0, n)])

    def load(j, buf):
        pltpu.async_copy(ea_hbm.at[wid, pl.ds(j * KE, KE)], buf, semg)

    _dbuf_loop(KE, CHE, load, stage_idx, dst_v, acce, ze_hbm,
               r0, r1, semg, sems)
    plsc.subcore_barrier()
    _writeback(acce, oute_hbm, c, s)


@functools.cache
def _build_sc_edge_sum():
    return pl.kernel(
        _sc_edge_sum_body,
        out_type=jax.ShapeDtypeStruct((NC, N_NODES, DEA), _f32),
        mesh=_get_mesh(),
        scratch_types=[
            pltpu.VMEM((SLAB, KE), jnp.int32),
            pltpu.VMEM((KE, DEA), _f32),
            pltpu.VMEM((KE, DEA), _f32),
            pltpu.VMEM_SHARED((NROWS, DEA), _f32),
            pltpu.SemaphoreType.DMA,
            pltpu.SemaphoreType.DMA,
        ],
    )


def _dot(a, b):
    return lax.dot_general(a, b, (((1,), (0,)), ((), ())),
                           precision=lax.Precision.HIGHEST,
                           preferred_element_type=_f32)


def _tc_layer_kernel(gp_ref, ep_ref, hin_ref, wx_ref, we_ref, bn_ref,
                     wr_ref, br_ref, g_ref, b_ref, out_ref):
    gsum = gp_ref[0] + gp_ref[1]
    esum = ep_ref[0] + ep_ref[1]
    cnt = esum[:, D_EDGE:D_EDGE + 1]
    msum = (_dot(gsum, wx_ref[0]) + _dot(esum[:, :D_EDGE], we_ref[0])
            + cnt * bn_ref[...])
    agg = msum / jnp.maximum(cnt, 1.0)
    h = agg + _dot(hin_ref[...], wr_ref[0]) + br_ref[...]
    h = jnp.maximum(h, 0.0)
    mu = jnp.mean(h, axis=-1, keepdims=True)
    var = jnp.mean((h - mu) ** 2, axis=-1, keepdims=True)
    out_ref[...] = (h - mu) * lax.rsqrt(var + 1e-5) * g_ref[...] + b_ref[...]


def _tc_pool_kernel(h_ref, wp_ref, bp_ref, ws_ref, bs_ref, batch_ref,
                    out_ref):
    h = h_ref[...]
    hp = jnp.tanh(_dot(h, wp_ref[0]) + bp_ref[...])
    s = _dot(hp, ws_ref[...]) + bs_ref[...]
    gid = lax.broadcasted_iota(jnp.int32, (N_NODES, NUM_GRAPHS), 1)
    oh = (batch_ref[...] == gid)
    ohf = oh.astype(_f32)
    smax = jnp.max(jnp.where(oh, s, -1e30), axis=0, keepdims=True)
    smax_b = jnp.sum(jnp.where(oh, smax, 0.0), axis=1, keepdims=True)
    e = jnp.exp(s - smax_b)
    denom = jnp.sum(ohf * e, axis=0, keepdims=True)
    denom_b = jnp.sum(ohf * denom, axis=1, keepdims=True)
    w = e / denom_b
    out_ref[...] = lax.dot_general(
        ohf * w, h, (((0,), (0,)), ((), ())),
        precision=lax.Precision.HIGHEST, preferred_element_type=_f32)


_RB = 1000
_NRB = N_NODES // _RB


def _tc_layer(gp, ep, hin, wn, bn, wr, br, g, b):
    return pl.pallas_call(
        _tc_layer_kernel,
        grid=(_NRB,),
        in_specs=[
            pl.BlockSpec((NC, _RB, D_HID), lambda i: (0, i, 0)),
            pl.BlockSpec((NC, _RB, DEA), lambda i: (0, i, 0)),
            pl.BlockSpec((_RB, D_HID), lambda i: (i, 0)),
            pl.BlockSpec((1, D_HID, D_HID), lambda i: (0, 0, 0)),
            pl.BlockSpec((1, D_EDGE, D_HID), lambda i: (0, 0, 0)),
            pl.BlockSpec((1, D_HID), lambda i: (0, 0)),
            pl.BlockSpec((1, D_HID, D_HID), lambda i: (0, 0, 0)),
            pl.BlockSpec((1, D_HID), lambda i: (0, 0)),
            pl.BlockSpec((1, D_HID), lambda i: (0, 0)),
            pl.BlockSpec((1, D_HID), lambda i: (0, 0)),
        ],
        out_specs=pl.BlockSpec((_RB, D_HID), lambda i: (i, 0)),
        out_shape=jax.ShapeDtypeStruct((N_NODES, D_HID), _f32),
    )(gp, ep, hin,
      wn[:D_HID].reshape(1, D_HID, D_HID),
      wn[D_HID:].reshape(1, D_EDGE, D_HID),
      bn.reshape(1, D_HID),
      wr.reshape(1, D_HID, D_HID),
      br.reshape(1, D_HID),
      g.reshape(1, D_HID), b.reshape(1, D_HID))


def _pad_idx(idx, ch, kk, trash):
    n_pad = ch * kk - EPT
    body = idx.reshape(NT, EPT)
    if trash:
        fill = (N_NODES + jnp.arange(NT, dtype=jnp.int32))[:, None]
        fill = jnp.broadcast_to(fill, (NT, n_pad))
    else:
        fill = jnp.zeros((NT, n_pad), jnp.int32)
    return jnp.concatenate([body, fill], axis=1).reshape(NT, ch, kk)


def kernel(x, edge_index, edge_attr, batch,
           W_neigh0, b_neigh0, W_root0, b_root0, ln_g0, ln_b0,
           W_neigh1, b_neigh1, W_root1, b_root1, ln_g1, ln_b1,
           W_proj, b_proj, w_score, b_score):
    src_g = edge_index[0].reshape(NT, CHG, KG)
    dst_g = edge_index[1].reshape(NT, CHG, KG)
    dst_e = _pad_idx(edge_index[1], CHE, KE, trash=True)
    ea_pad = jnp.concatenate(
        [edge_attr.reshape(NT, EPT, D_EDGE),
         jnp.ones((NT, EPT, 1), _f32),
         jnp.zeros((NT, EPT, DEA - D_EDGE - 1), _f32)], axis=2)
    ea_pad = jnp.concatenate(
        [ea_pad, jnp.zeros((NT, CHE * KE - EPT, DEA), _f32)], axis=1)
    zg = jnp.zeros((RPT, D_HID), _f32)

    gather_sum = _build_sc_gather_sum()
    ep = _build_sc_edge_sum()(ea_pad, dst_e, zg)
    gp0 = gather_sum(x, src_g, dst_g, zg)
    h0 = _tc_layer(gp0, ep, x, W_neigh0, b_neigh0, W_root0, b_root0,
                   ln_g0, ln_b0)
    gp1 = gather_sum(h0, src_g, dst_g, zg)
    h1 = _tc_layer(gp1, ep, h0, W_neigh1, b_neigh1, W_root1, b_root1,
                   ln_g1, ln_b1)

    pooled = pl.pallas_call(
        _tc_pool_kernel,
        out_shape=jax.ShapeDtypeStruct((NUM_GRAPHS, D_HID), _f32),
    )(h1, W_proj.reshape(1, D_HID, D_HID), b_proj.reshape(1, D_HID),
      w_score.reshape(D_HID, 1), b_score.reshape(1, 1),
      batch.reshape(N_NODES, 1))
    return pooled
